# baseline (speedup 1.0000x reference)
import functools

import jax
import jax.numpy as jnp
from jax import lax
from jax.experimental import pallas as pl
from jax.experimental.pallas import tpu as pltpu
from jax.experimental.pallas import tpu_sc as plsc

N = 4096
C_IN = 128
D = 256
K = 32
EPS = 1e-5

BM = 256
NT = 8
TCOL = N // NT
NBLK = N // BM

NW = 32
NODES_PER_W = N // NW
NODES_PER_CHUNK = 4
CHUNKS_PER_W = NODES_PER_W // NODES_PER_CHUNK
IDX_PER_CHUNK = NODES_PER_CHUNK * K


def _proj1_body(xf_ref, waug_ref, bb_ref, wr_ref, wn_ref,
                feat_ref, xr_ref, xn_ref, sq_ref):
    f = jnp.dot(xf_ref[...], waug_ref[...],
                preferred_element_type=jnp.float32) + bb_ref[...]
    feat_ref[...] = f
    xr_ref[...] = lax.dot_general(f, wr_ref[...], (((1,), (1,)), ((), ())),
                                  preferred_element_type=jnp.float32)
    xn_ref[...] = lax.dot_general(f, wn_ref[...], (((1,), (1,)), ((), ())),
                                  preferred_element_type=jnp.float32)
    sq_ref[...] = jnp.sum(f * f, axis=1, keepdims=True)


def _proj2_body(f_ref, wr_ref, wn_ref, xr_ref, xn_ref, sq_ref):
    f = f_ref[...]
    xr_ref[...] = lax.dot_general(f, wr_ref[...], (((1,), (1,)), ((), ())),
                                  preferred_element_type=jnp.float32)
    xn_ref[...] = lax.dot_general(f, wn_ref[...], (((1,), (1,)), ((), ())),
                                  preferred_element_type=jnp.float32)
    sq_ref[...] = jnp.sum(f * f, axis=1, keepdims=True)


def _knn_body(fb_ref, ff_ref, sqr_ref, idx_ref, s_ref):
    fb = fb_ref[...]
    for c in range(NT):
        g = lax.dot_general(fb, ff_ref[c * TCOL:(c + 1) * TCOL, :],
                            (((1,), (1,)), ((), ())),
                            precision=lax.Precision.HIGHEST,
                            preferred_element_type=jnp.float32)
        s_ref[c] = 2.0 * g - sqr_ref[c]

    kiota = lax.broadcasted_iota(jnp.int32, (1, K), 1)
    tiota = lax.broadcasted_iota(jnp.int32, (1, TCOL), 1)
    neg_inf = jnp.float32(-jnp.inf)

    def lexmerge(a, b):
        m1, j1 = a
        m2, j2 = b
        m = jnp.maximum(m1, m2)
        j = jnp.where(m1 > m2, j1,
                      jnp.where(m2 > m1, j2, jnp.minimum(j1, j2)))
        return (m, j)

    def step(t, carry):
        J, jprev = carry
        pairs = []
        for c in range(NT):
            tile = s_ref[c]
            ii = tiota + c * TCOL
            masked = jnp.where(ii == jprev, neg_inf, tile)
            s_ref[c] = masked
            tmax = jnp.max(masked, axis=1, keepdims=True)
            tj = jnp.min(jnp.where(masked == tmax, ii, N), axis=1,
                         keepdims=True)
            pairs.append((tmax, tj))
        while len(pairs) > 1:
            pairs = [lexmerge(pairs[i], pairs[i + 1])
                     for i in range(0, len(pairs), 2)]
        _, j = pairs[0]
        return (jnp.where(kiota == t, j, J), j)

    J, _ = lax.fori_loop(
        0, K, step,
        (jnp.zeros((BM, K), dtype=jnp.int32),
         jnp.full((BM, 1), -1, dtype=jnp.int32)))
    idx_ref[...] = J


def _proj1(xf, waug, bb, wr, wn):
    return pl.pallas_call(
        _proj1_body,
        grid=(8,),
        in_specs=[
            pl.BlockSpec((N // 8, C_IN * 16), lambda b: (b, 0)),
            pl.BlockSpec((C_IN * 16, D), lambda b: (0, 0)),
            pl.BlockSpec((1, D), lambda b: (0, 0)),
            pl.BlockSpec((D, D), lambda b: (0, 0)),
            pl.BlockSpec((D, D), lambda b: (0, 0)),
        ],
        out_specs=[
            pl.BlockSpec((N // 8, D), lambda b: (b, 0)),
            pl.BlockSpec((N // 8, D), lambda b: (b, 0)),
            pl.BlockSpec((N // 8, D), lambda b: (b, 0)),
            pl.BlockSpec((N // 8, 1), lambda b: (b, 0)),
        ],
        out_shape=[
            jax.ShapeDtypeStruct((N, D), jnp.float32),
            jax.ShapeDtypeStruct((N, D), jnp.float32),
            jax.ShapeDtypeStruct((N, D), jnp.float32),
            jax.ShapeDtypeStruct((N, 1), jnp.float32),
        ],
    )(xf, waug, bb, wr, wn)


def _proj2(f, wr, wn):
    return pl.pallas_call(
        _proj2_body,
        grid=(8,),
        in_specs=[
            pl.BlockSpec((N // 8, D), lambda b: (b, 0)),
            pl.BlockSpec((D, D), lambda b: (0, 0)),
            pl.BlockSpec((D, D), lambda b: (0, 0)),
        ],
        out_specs=[
            pl.BlockSpec((N // 8, D), lambda b: (b, 0)),
            pl.BlockSpec((N // 8, D), lambda b: (b, 0)),
            pl.BlockSpec((N // 8, 1), lambda b: (b, 0)),
        ],
        out_shape=[
            jax.ShapeDtypeStruct((N, D), jnp.float32),
            jax.ShapeDtypeStruct((N, D), jnp.float32),
            jax.ShapeDtypeStruct((N, 1), jnp.float32),
        ],
    )(f, wr, wn)


def _knn(feat, sq3):
    return pl.pallas_call(
        _knn_body,
        grid=(NBLK,),
        in_specs=[
            pl.BlockSpec((BM, D), lambda b: (b, 0)),
            pl.BlockSpec((N, D), lambda b: (0, 0)),
            pl.BlockSpec((NT, 1, TCOL), lambda b: (0, 0, 0)),
        ],
        out_specs=pl.BlockSpec((BM, K), lambda b: (b, 0)),
        out_shape=jax.ShapeDtypeStruct((N, K), jnp.int32),
        scratch_shapes=[pltpu.VMEM((NT, BM, TCOL), jnp.float32)],
    )(feat, feat, sq3)


def _agg_sc_body(xn_hbm, xr_hbm, idx_hbm, scale_hbm, shift_hbm, out_hbm,
                 idx_v, rows_v, out_v, xr_v, scale_v, shift_v, sem):
    wid = lax.axis_index("s") * 2 + lax.axis_index("c")
    base = wid * NODES_PER_W
    pltpu.sync_copy(idx_hbm.at[wid], idx_v)
    pltpu.sync_copy(scale_hbm, scale_v)
    pltpu.sync_copy(shift_hbm, shift_v)

    def chunk_body(c, _):
        pltpu.async_copy(xn_hbm.at[idx_v.at[c]], rows_v, sem).wait()
        row0 = base + c * NODES_PER_CHUNK
        pltpu.sync_copy(xr_hbm.at[pl.ds(row0, NODES_PER_CHUNK)], xr_v)
        for n in range(NODES_PER_CHUNK):
            def rbody(r, accs):
                return tuple(accs[dd] + rows_v[n * K + r, pl.ds(dd * 16, 16)]
                             for dd in range(16))
            acc0 = tuple(rows_v[n * K, pl.ds(dd * 16, 16)] for dd in range(16))
            accs = lax.fori_loop(1, K, rbody, acc0)
            for dd in range(16):
                sl = pl.ds(dd * 16, 16)
                out_v[n, sl] = ((xr_v[n, sl] + accs[dd]) * scale_v[0, sl]
                                + shift_v[0, sl])
        pltpu.sync_copy(out_v, out_hbm.at[pl.ds(row0, NODES_PER_CHUNK)])
        return 0

    lax.fori_loop(0, CHUNKS_PER_W, chunk_body, 0)


def _agg_sc(xn, xr, idx3, scale, shift):
    mesh = plsc.VectorSubcoreMesh(core_axis_name="c", subcore_axis_name="s")
    k = functools.partial(
        pl.kernel,
        out_type=jax.ShapeDtypeStruct((N, D), jnp.float32),
        mesh=mesh,
        scratch_types=[
            pltpu.VMEM((CHUNKS_PER_W, IDX_PER_CHUNK), jnp.int32),
            pltpu.VMEM((IDX_PER_CHUNK, D), jnp.float32),
            pltpu.VMEM((NODES_PER_CHUNK, D), jnp.float32),
            pltpu.VMEM((NODES_PER_CHUNK, D), jnp.float32),
            pltpu.VMEM((1, D), jnp.float32),
            pltpu.VMEM((1, D), jnp.float32),
            pltpu.SemaphoreType.DMA,
        ],
    )(_agg_sc_body)
    return k(xn, xr, idx3, scale, shift)


def kernel(x, W_base, b_base, W1_root, W1_nbr, b1, bn_gamma, bn_beta,
           W2_root, W2_nbr, b2):
    waug = jnp.repeat(jnp.transpose(W_base)[:, None, :] / 16.0, 16,
                      axis=1).reshape(C_IN * 16, D)
    xf = x.reshape(N, C_IN * 16)
    bb = b_base.reshape(1, D)
    gp = bn_gamma / jnp.sqrt(1.0 + EPS)
    scale1 = gp.reshape(1, D)
    shift1 = (b1 * gp + bn_beta).reshape(1, D)
    scale2 = jnp.ones((1, D), jnp.float32)
    shift2 = b2.reshape(1, D)

    feat, xr1, xn1, sq1 = _proj1(xf, waug, bb, W1_root, W1_nbr)
    idx1 = _knn(feat, sq1.reshape(NT, 1, TCOL))
    f2 = _agg_sc(xn1, xr1, idx1.reshape(NW, CHUNKS_PER_W, IDX_PER_CHUNK),
                 scale1, shift1)
    xr2, xn2, sq2 = _proj2(f2, W2_root, W2_nbr)
    idx2 = _knn(f2, sq2.reshape(NT, 1, TCOL))
    out = _agg_sc(xn2, xr2, idx2.reshape(NW, CHUNKS_PER_W, IDX_PER_CHUNK),
                  scale2, shift2)
    return out


# speedup vs baseline: 1.0775x; 1.0775x over previous
import functools

import jax
import jax.numpy as jnp
from jax import lax
from jax.experimental import pallas as pl
from jax.experimental.pallas import tpu as pltpu
from jax.experimental.pallas import tpu_sc as plsc

N = 4096
C_IN = 128
D = 256
K = 32
EPS = 1e-5

BM = 256
NT = 8
TCOL = N // NT
NBLK = N // BM

NW = 32
NODES_PER_W = N // NW
NODES_PER_CHUNK = 4
CHUNKS_PER_W = NODES_PER_W // NODES_PER_CHUNK
IDX_PER_CHUNK = NODES_PER_CHUNK * K


def _proj1_body(xf_ref, waug_ref, bb_ref, wr_ref, wn_ref,
                feat_ref, xr_ref, xn_ref, sq_ref):
    f = jnp.dot(xf_ref[...], waug_ref[...],
                preferred_element_type=jnp.float32) + bb_ref[...]
    feat_ref[...] = f
    xr_ref[...] = lax.dot_general(f, wr_ref[...], (((1,), (1,)), ((), ())),
                                  preferred_element_type=jnp.float32)
    xn_ref[...] = lax.dot_general(f, wn_ref[...], (((1,), (1,)), ((), ())),
                                  preferred_element_type=jnp.float32)
    sq_ref[...] = jnp.sum(f * f, axis=1, keepdims=True)


def _proj2_body(f_ref, wr_ref, wn_ref, xr_ref, xn_ref, sq_ref):
    f = f_ref[...]
    xr_ref[...] = lax.dot_general(f, wr_ref[...], (((1,), (1,)), ((), ())),
                                  preferred_element_type=jnp.float32)
    xn_ref[...] = lax.dot_general(f, wn_ref[...], (((1,), (1,)), ((), ())),
                                  preferred_element_type=jnp.float32)
    sq_ref[...] = jnp.sum(f * f, axis=1, keepdims=True)


def _knn_body(fb_ref, ff_ref, sqr_ref, idx_ref, s_ref):
    fb = fb_ref[...]
    for c in range(NT):
        g = lax.dot_general(fb, ff_ref[c * TCOL:(c + 1) * TCOL, :],
                            (((1,), (1,)), ((), ())),
                            preferred_element_type=jnp.float32)
        s_ref[c] = 2.0 * g - sqr_ref[c]

    kiota = lax.broadcasted_iota(jnp.int32, (1, K), 1)
    tiota = lax.broadcasted_iota(jnp.int32, (1, TCOL), 1)
    neg_inf = jnp.float32(-jnp.inf)

    def lexmerge(a, b):
        m1, j1 = a
        m2, j2 = b
        m = jnp.maximum(m1, m2)
        j = jnp.where(m1 > m2, j1,
                      jnp.where(m2 > m1, j2, jnp.minimum(j1, j2)))
        return (m, j)

    def step(t, carry):
        J, jprev = carry
        pairs = []
        for c in range(NT):
            tile = s_ref[c]
            ii = tiota + c * TCOL
            masked = jnp.where(ii == jprev, neg_inf, tile)
            s_ref[c] = masked
            tmax = jnp.max(masked, axis=1, keepdims=True)
            tj = jnp.min(jnp.where(masked == tmax, ii, N), axis=1,
                         keepdims=True)
            pairs.append((tmax, tj))
        while len(pairs) > 1:
            pairs = [lexmerge(pairs[i], pairs[i + 1])
                     for i in range(0, len(pairs), 2)]
        _, j = pairs[0]
        return (jnp.where(kiota == t, j, J), j)

    J, _ = lax.fori_loop(
        0, K, step,
        (jnp.zeros((BM, K), dtype=jnp.int32),
         jnp.full((BM, 1), -1, dtype=jnp.int32)))
    idx_ref[...] = J


def _proj1(xf, waug, bb, wr, wn):
    return pl.pallas_call(
        _proj1_body,
        grid=(8,),
        in_specs=[
            pl.BlockSpec((N // 8, C_IN * 16), lambda b: (b, 0)),
            pl.BlockSpec((C_IN * 16, D), lambda b: (0, 0)),
            pl.BlockSpec((1, D), lambda b: (0, 0)),
            pl.BlockSpec((D, D), lambda b: (0, 0)),
            pl.BlockSpec((D, D), lambda b: (0, 0)),
        ],
        out_specs=[
            pl.BlockSpec((N // 8, D), lambda b: (b, 0)),
            pl.BlockSpec((N // 8, D), lambda b: (b, 0)),
            pl.BlockSpec((N // 8, D), lambda b: (b, 0)),
            pl.BlockSpec((N // 8, 1), lambda b: (b, 0)),
        ],
        out_shape=[
            jax.ShapeDtypeStruct((N, D), jnp.float32),
            jax.ShapeDtypeStruct((N, D), jnp.float32),
            jax.ShapeDtypeStruct((N, D), jnp.float32),
            jax.ShapeDtypeStruct((N, 1), jnp.float32),
        ],
    )(xf, waug, bb, wr, wn)


def _proj2(f, wr, wn):
    return pl.pallas_call(
        _proj2_body,
        grid=(8,),
        in_specs=[
            pl.BlockSpec((N // 8, D), lambda b: (b, 0)),
            pl.BlockSpec((D, D), lambda b: (0, 0)),
            pl.BlockSpec((D, D), lambda b: (0, 0)),
        ],
        out_specs=[
            pl.BlockSpec((N // 8, D), lambda b: (b, 0)),
            pl.BlockSpec((N // 8, D), lambda b: (b, 0)),
            pl.BlockSpec((N // 8, 1), lambda b: (b, 0)),
        ],
        out_shape=[
            jax.ShapeDtypeStruct((N, D), jnp.float32),
            jax.ShapeDtypeStruct((N, D), jnp.float32),
            jax.ShapeDtypeStruct((N, 1), jnp.float32),
        ],
    )(f, wr, wn)


def _knn(feat, sq3):
    return pl.pallas_call(
        _knn_body,
        grid=(NBLK,),
        in_specs=[
            pl.BlockSpec((BM, D), lambda b: (b, 0)),
            pl.BlockSpec((N, D), lambda b: (0, 0)),
            pl.BlockSpec((NT, 1, TCOL), lambda b: (0, 0, 0)),
        ],
        out_specs=pl.BlockSpec((BM, K), lambda b: (b, 0)),
        out_shape=jax.ShapeDtypeStruct((N, K), jnp.int32),
        scratch_shapes=[pltpu.VMEM((NT, BM, TCOL), jnp.float32)],
    )(feat, feat, sq3)


def _agg_sc_body(xn_hbm, xr_hbm, idx_hbm, scale_hbm, shift_hbm, out_hbm,
                 idx_v, rows_a, rows_b, out_v, xr_v, scale_v, shift_v,
                 sem_a, sem_b):
    wid = lax.axis_index("s") * 2 + lax.axis_index("c")
    base = wid * NODES_PER_W
    pltpu.sync_copy(idx_hbm.at[wid], idx_v)
    pltpu.sync_copy(scale_hbm, scale_v)
    pltpu.sync_copy(shift_hbm, shift_v)

    def start(c, buf, sem):
        pltpu.async_copy(xn_hbm.at[idx_v.at[c]], buf, sem)

    def drain(buf, sem):
        pltpu.make_async_copy(
            xn_hbm.at[pl.ds(0, IDX_PER_CHUNK)], buf, sem).wait()

    def compute(c, buf):
        row0 = base + c * NODES_PER_CHUNK
        pltpu.sync_copy(xr_hbm.at[pl.ds(row0, NODES_PER_CHUNK)], xr_v)
        for n in range(NODES_PER_CHUNK):
            def rbody(r, accs):
                return tuple(accs[dd] + buf[n * K + r, pl.ds(dd * 16, 16)]
                             for dd in range(16))
            acc0 = tuple(buf[n * K, pl.ds(dd * 16, 16)] for dd in range(16))
            accs = lax.fori_loop(1, K, rbody, acc0)
            for dd in range(16):
                sl = pl.ds(dd * 16, 16)
                out_v[n, sl] = ((xr_v[n, sl] + accs[dd]) * scale_v[0, sl]
                                + shift_v[0, sl])
        pltpu.sync_copy(out_v, out_hbm.at[pl.ds(row0, NODES_PER_CHUNK)])

    start(0, rows_a, sem_a)

    def pair_body(i, _):
        c0 = 2 * i
        start(c0 + 1, rows_b, sem_b)
        drain(rows_a, sem_a)
        compute(c0, rows_a)

        @pl.when(c0 + 2 < CHUNKS_PER_W)
        def _():
            start(c0 + 2, rows_a, sem_a)

        drain(rows_b, sem_b)
        compute(c0 + 1, rows_b)
        return 0

    lax.fori_loop(0, CHUNKS_PER_W // 2, pair_body, 0)


def _agg_sc(xn, xr, idx3, scale, shift):
    mesh = plsc.VectorSubcoreMesh(core_axis_name="c", subcore_axis_name="s")
    k = functools.partial(
        pl.kernel,
        out_type=jax.ShapeDtypeStruct((N, D), jnp.float32),
        mesh=mesh,
        scratch_types=[
            pltpu.VMEM((CHUNKS_PER_W, IDX_PER_CHUNK), jnp.int32),
            pltpu.VMEM((IDX_PER_CHUNK, D), jnp.float32),
            pltpu.VMEM((IDX_PER_CHUNK, D), jnp.float32),
            pltpu.VMEM((NODES_PER_CHUNK, D), jnp.float32),
            pltpu.VMEM((NODES_PER_CHUNK, D), jnp.float32),
            pltpu.VMEM((1, D), jnp.float32),
            pltpu.VMEM((1, D), jnp.float32),
            pltpu.SemaphoreType.DMA,
            pltpu.SemaphoreType.DMA,
        ],
    )(_agg_sc_body)
    return k(xn, xr, idx3, scale, shift)


def kernel(x, W_base, b_base, W1_root, W1_nbr, b1, bn_gamma, bn_beta,
           W2_root, W2_nbr, b2):
    waug = jnp.repeat(jnp.transpose(W_base)[:, None, :] / 16.0, 16,
                      axis=1).reshape(C_IN * 16, D)
    xf = x.reshape(N, C_IN * 16)
    bb = b_base.reshape(1, D)
    gp = bn_gamma / jnp.sqrt(1.0 + EPS)
    scale1 = gp.reshape(1, D)
    shift1 = (b1 * gp + bn_beta).reshape(1, D)
    scale2 = jnp.ones((1, D), jnp.float32)
    shift2 = b2.reshape(1, D)

    feat, xr1, xn1, sq1 = _proj1(xf, waug, bb, W1_root, W1_nbr)
    idx1 = _knn(feat, sq1.reshape(NT, 1, TCOL))
    f2 = _agg_sc(xn1, xr1, idx1.reshape(NW, CHUNKS_PER_W, IDX_PER_CHUNK),
                 scale1, shift1)
    xr2, xn2, sq2 = _proj2(f2, W2_root, W2_nbr)
    idx2 = _knn(f2, sq2.reshape(NT, 1, TCOL))
    out = _agg_sc(xn2, xr2, idx2.reshape(NW, CHUNKS_PER_W, IDX_PER_CHUNK),
                  scale2, shift2)
    return out


# speedup vs baseline: 1.3202x; 1.2252x over previous
import functools

import jax
import jax.numpy as jnp
from jax import lax
from jax.experimental import pallas as pl
from jax.experimental.pallas import tpu as pltpu
from jax.experimental.pallas import tpu_sc as plsc

N = 4096
C_IN = 128
D = 256
K = 32
EPS = 1e-5

BM = 512
NT = 8
TCOL = N // NT
NBLK = N // BM

NW = 32
NODES_PER_W = N // NW
NODES_PER_CHUNK = 4
CHUNKS_PER_W = NODES_PER_W // NODES_PER_CHUNK
IDX_PER_CHUNK = NODES_PER_CHUNK * K


def _proj1_body(xf_ref, waug_ref, bb_ref, wr_ref, wn_ref,
                feat_ref, xr_ref, xn_ref, sq_ref):
    f = jnp.dot(xf_ref[...], waug_ref[...],
                preferred_element_type=jnp.float32) + bb_ref[...]
    feat_ref[...] = f
    xr_ref[...] = lax.dot_general(f, wr_ref[...], (((1,), (1,)), ((), ())),
                                  preferred_element_type=jnp.float32)
    xn_ref[...] = lax.dot_general(f, wn_ref[...], (((1,), (1,)), ((), ())),
                                  preferred_element_type=jnp.float32)
    sq_ref[...] = jnp.sum(f * f, axis=1, keepdims=True)


def _proj2_body(f_ref, wr_ref, wn_ref, xr_ref, xn_ref, sq_ref):
    f = f_ref[...]
    xr_ref[...] = lax.dot_general(f, wr_ref[...], (((1,), (1,)), ((), ())),
                                  preferred_element_type=jnp.float32)
    xn_ref[...] = lax.dot_general(f, wn_ref[...], (((1,), (1,)), ((), ())),
                                  preferred_element_type=jnp.float32)
    sq_ref[...] = jnp.sum(f * f, axis=1, keepdims=True)


def _knn_body(fb_ref, ff_ref, sqr_ref, idx_ref, s_ref):
    fb = fb_ref[...]
    for c in range(NT):
        g = lax.dot_general(fb, ff_ref[c * TCOL:(c + 1) * TCOL, :],
                            (((1,), (1,)), ((), ())),
                            preferred_element_type=jnp.float32)
        s_ref[c] = 2.0 * g - sqr_ref[c]

    kiota = lax.broadcasted_iota(jnp.int32, (1, K), 1)
    tiota = lax.broadcasted_iota(jnp.int32, (1, TCOL), 1)
    neg_inf = jnp.float32(-jnp.inf)

    def lexmerge(a, b):
        m1, j1 = a
        m2, j2 = b
        m = jnp.maximum(m1, m2)
        j = jnp.where(m1 > m2, j1,
                      jnp.where(m2 > m1, j2, jnp.minimum(j1, j2)))
        return (m, j)

    def step(t, carry):
        J, jprev = carry
        pairs = []
        for c in range(NT):
            tile = s_ref[c]
            ii = tiota + c * TCOL
            masked = jnp.where(ii == jprev, neg_inf, tile)
            s_ref[c] = masked
            tmax = jnp.max(masked, axis=1, keepdims=True)
            tj = jnp.min(jnp.where(masked == tmax, ii, N), axis=1,
                         keepdims=True)
            pairs.append((tmax, tj))
        while len(pairs) > 1:
            pairs = [lexmerge(pairs[i], pairs[i + 1])
                     for i in range(0, len(pairs), 2)]
        _, j = pairs[0]
        return (jnp.where(kiota == t, j, J), j)

    J, _ = lax.fori_loop(
        0, K, step,
        (jnp.zeros((BM, K), dtype=jnp.int32),
         jnp.full((BM, 1), -1, dtype=jnp.int32)))
    idx_ref[...] = J


def _proj1(xf, waug, bb, wr, wn):
    return pl.pallas_call(
        _proj1_body,
        grid=(8,),
        in_specs=[
            pl.BlockSpec((N // 8, C_IN * 16), lambda b: (b, 0)),
            pl.BlockSpec((C_IN * 16, D), lambda b: (0, 0)),
            pl.BlockSpec((1, D), lambda b: (0, 0)),
            pl.BlockSpec((D, D), lambda b: (0, 0)),
            pl.BlockSpec((D, D), lambda b: (0, 0)),
        ],
        out_specs=[
            pl.BlockSpec((N // 8, D), lambda b: (b, 0)),
            pl.BlockSpec((N // 8, D), lambda b: (b, 0)),
            pl.BlockSpec((N // 8, D), lambda b: (b, 0)),
            pl.BlockSpec((N // 8, 1), lambda b: (b, 0)),
        ],
        out_shape=[
            jax.ShapeDtypeStruct((N, D), jnp.float32),
            jax.ShapeDtypeStruct((N, D), jnp.float32),
            jax.ShapeDtypeStruct((N, D), jnp.float32),
            jax.ShapeDtypeStruct((N, 1), jnp.float32),
        ],
    )(xf, waug, bb, wr, wn)


def _proj2(f, wr, wn):
    return pl.pallas_call(
        _proj2_body,
        grid=(8,),
        in_specs=[
            pl.BlockSpec((N // 8, D), lambda b: (b, 0)),
            pl.BlockSpec((D, D), lambda b: (0, 0)),
            pl.BlockSpec((D, D), lambda b: (0, 0)),
        ],
        out_specs=[
            pl.BlockSpec((N // 8, D), lambda b: (b, 0)),
            pl.BlockSpec((N // 8, D), lambda b: (b, 0)),
            pl.BlockSpec((N // 8, 1), lambda b: (b, 0)),
        ],
        out_shape=[
            jax.ShapeDtypeStruct((N, D), jnp.float32),
            jax.ShapeDtypeStruct((N, D), jnp.float32),
            jax.ShapeDtypeStruct((N, 1), jnp.float32),
        ],
    )(f, wr, wn)


def _knn(feat, sq3):
    return pl.pallas_call(
        _knn_body,
        grid=(NBLK,),
        in_specs=[
            pl.BlockSpec((BM, D), lambda b: (b, 0)),
            pl.BlockSpec((N, D), lambda b: (0, 0)),
            pl.BlockSpec((NT, 1, TCOL), lambda b: (0, 0, 0)),
        ],
        out_specs=pl.BlockSpec((BM, K), lambda b: (b, 0)),
        out_shape=jax.ShapeDtypeStruct((N, K), jnp.int32),
        scratch_shapes=[pltpu.VMEM((NT, BM, TCOL), jnp.float32)],
    )(feat, feat, sq3)


def _agg_sc_body(xn_hbm, xr_hbm, idx_hbm, scale_hbm, shift_hbm, out_hbm,
                 idx_v, rows_a, rows_b, out_v, xr_v, scale_v, shift_v,
                 sem_a, sem_b):
    wid = lax.axis_index("s") * 2 + lax.axis_index("c")
    base = wid * NODES_PER_W
    pltpu.sync_copy(idx_hbm.at[wid], idx_v)
    pltpu.sync_copy(scale_hbm, scale_v)
    pltpu.sync_copy(shift_hbm, shift_v)

    def start(c, buf, sem):
        pltpu.async_copy(xn_hbm.at[idx_v.at[c]], buf, sem)

    def drain(buf, sem):
        pltpu.make_async_copy(
            xn_hbm.at[pl.ds(0, IDX_PER_CHUNK)], buf, sem).wait()

    def compute(c, buf):
        row0 = base + c * NODES_PER_CHUNK
        pltpu.sync_copy(xr_hbm.at[pl.ds(row0, NODES_PER_CHUNK)], xr_v)
        for n in range(NODES_PER_CHUNK):
            def rbody(r, accs):
                return tuple(accs[dd] + buf[n * K + r, pl.ds(dd * 16, 16)]
                             for dd in range(16))
            acc0 = tuple(buf[n * K, pl.ds(dd * 16, 16)] for dd in range(16))
            accs = lax.fori_loop(1, K, rbody, acc0)
            for dd in range(16):
                sl = pl.ds(dd * 16, 16)
                out_v[n, sl] = ((xr_v[n, sl] + accs[dd]) * scale_v[0, sl]
                                + shift_v[0, sl])
        pltpu.sync_copy(out_v, out_hbm.at[pl.ds(row0, NODES_PER_CHUNK)])

    start(0, rows_a, sem_a)

    def pair_body(i, _):
        c0 = 2 * i
        start(c0 + 1, rows_b, sem_b)
        drain(rows_a, sem_a)
        compute(c0, rows_a)

        @pl.when(c0 + 2 < CHUNKS_PER_W)
        def _():
            start(c0 + 2, rows_a, sem_a)

        drain(rows_b, sem_b)
        compute(c0 + 1, rows_b)
        return 0

    lax.fori_loop(0, CHUNKS_PER_W // 2, pair_body, 0)


def _agg_sc(xn, xr, idx3, scale, shift):
    mesh = plsc.VectorSubcoreMesh(core_axis_name="c", subcore_axis_name="s")
    k = functools.partial(
        pl.kernel,
        out_type=jax.ShapeDtypeStruct((N, D), jnp.float32),
        mesh=mesh,
        scratch_types=[
            pltpu.VMEM((CHUNKS_PER_W, IDX_PER_CHUNK), jnp.int32),
            pltpu.VMEM((IDX_PER_CHUNK, D), jnp.float32),
            pltpu.VMEM((IDX_PER_CHUNK, D), jnp.float32),
            pltpu.VMEM((NODES_PER_CHUNK, D), jnp.float32),
            pltpu.VMEM((NODES_PER_CHUNK, D), jnp.float32),
            pltpu.VMEM((1, D), jnp.float32),
            pltpu.VMEM((1, D), jnp.float32),
            pltpu.SemaphoreType.DMA,
            pltpu.SemaphoreType.DMA,
        ],
    )(_agg_sc_body)
    return k(xn, xr, idx3, scale, shift)


def kernel(x, W_base, b_base, W1_root, W1_nbr, b1, bn_gamma, bn_beta,
           W2_root, W2_nbr, b2):
    waug = jnp.repeat(jnp.transpose(W_base)[:, None, :] / 16.0, 16,
                      axis=1).reshape(C_IN * 16, D)
    xf = x.reshape(N, C_IN * 16)
    bb = b_base.reshape(1, D)
    gp = bn_gamma / jnp.sqrt(1.0 + EPS)
    scale1 = gp.reshape(1, D)
    shift1 = (b1 * gp + bn_beta).reshape(1, D)
    scale2 = jnp.ones((1, D), jnp.float32)
    shift2 = b2.reshape(1, D)

    feat, xr1, xn1, sq1 = _proj1(xf, waug, bb, W1_root, W1_nbr)
    idx1 = _knn(feat, sq1.reshape(NT, 1, TCOL))
    f2 = _agg_sc(xn1, xr1, idx1.reshape(NW, CHUNKS_PER_W, IDX_PER_CHUNK),
                 scale1, shift1)
    xr2, xn2, sq2 = _proj2(f2, W2_root, W2_nbr)
    idx2 = _knn(f2, sq2.reshape(NT, 1, TCOL))
    out = _agg_sc(xn2, xr2, idx2.reshape(NW, CHUNKS_PER_W, IDX_PER_CHUNK),
                  scale2, shift2)
    return out


# speedup vs baseline: 1.3347x; 1.0110x over previous
import functools

import jax
import jax.numpy as jnp
from jax import lax
from jax.experimental import pallas as pl
from jax.experimental.pallas import tpu as pltpu
from jax.experimental.pallas import tpu_sc as plsc

N = 4096
C_IN = 128
D = 256
K = 32
EPS = 1e-5

BM = 1024
NT = 8
TCOL = N // NT
NBLK = N // BM

NW = 32
NODES_PER_W = N // NW
NODES_PER_CHUNK = 4
CHUNKS_PER_W = NODES_PER_W // NODES_PER_CHUNK
IDX_PER_CHUNK = NODES_PER_CHUNK * K


def _proj1_body(xf_ref, waug_ref, bb_ref, wr_ref, wn_ref,
                feat_ref, xr_ref, xn_ref, sq_ref):
    f = jnp.dot(xf_ref[...], waug_ref[...],
                preferred_element_type=jnp.float32) + bb_ref[...]
    feat_ref[...] = f
    xr_ref[...] = lax.dot_general(f, wr_ref[...], (((1,), (1,)), ((), ())),
                                  preferred_element_type=jnp.float32)
    xn_ref[...] = lax.dot_general(f, wn_ref[...], (((1,), (1,)), ((), ())),
                                  preferred_element_type=jnp.float32)
    sq_ref[...] = jnp.sum(f * f, axis=1, keepdims=True)


def _proj2_body(f_ref, wr_ref, wn_ref, xr_ref, xn_ref, sq_ref):
    f = f_ref[...]
    xr_ref[...] = lax.dot_general(f, wr_ref[...], (((1,), (1,)), ((), ())),
                                  preferred_element_type=jnp.float32)
    xn_ref[...] = lax.dot_general(f, wn_ref[...], (((1,), (1,)), ((), ())),
                                  preferred_element_type=jnp.float32)
    sq_ref[...] = jnp.sum(f * f, axis=1, keepdims=True)


def _knn_body(fb_ref, ff_ref, sqr_ref, idx_ref, s_ref):
    fb = fb_ref[...]
    for c in range(NT):
        g = lax.dot_general(fb, ff_ref[c * TCOL:(c + 1) * TCOL, :],
                            (((1,), (1,)), ((), ())),
                            preferred_element_type=jnp.float32)
        s_ref[c] = 2.0 * g - sqr_ref[c]

    kiota = lax.broadcasted_iota(jnp.int32, (1, K), 1)
    tiota = lax.broadcasted_iota(jnp.int32, (1, TCOL), 1)
    neg_inf = jnp.float32(-jnp.inf)

    def lexmerge(a, b):
        m1, j1 = a
        m2, j2 = b
        m = jnp.maximum(m1, m2)
        j = jnp.where(m1 > m2, j1,
                      jnp.where(m2 > m1, j2, jnp.minimum(j1, j2)))
        return (m, j)

    def step(t, carry):
        J, jprev = carry
        pairs = []
        for c in range(NT):
            tile = s_ref[c]
            ii = tiota + c * TCOL
            masked = jnp.where(ii == jprev, neg_inf, tile)
            s_ref[c] = masked
            tmax = jnp.max(masked, axis=1, keepdims=True)
            tj = jnp.min(jnp.where(masked == tmax, ii, N), axis=1,
                         keepdims=True)
            pairs.append((tmax, tj))
        while len(pairs) > 1:
            pairs = [lexmerge(pairs[i], pairs[i + 1])
                     for i in range(0, len(pairs), 2)]
        _, j = pairs[0]
        return (jnp.where(kiota == t, j, J), j)

    J, _ = lax.fori_loop(
        0, K, step,
        (jnp.zeros((BM, K), dtype=jnp.int32),
         jnp.full((BM, 1), -1, dtype=jnp.int32)))
    idx_ref[...] = J


def _proj1(xf, waug, bb, wr, wn):
    return pl.pallas_call(
        _proj1_body,
        grid=(8,),
        in_specs=[
            pl.BlockSpec((N // 8, C_IN * 16), lambda b: (b, 0)),
            pl.BlockSpec((C_IN * 16, D), lambda b: (0, 0)),
            pl.BlockSpec((1, D), lambda b: (0, 0)),
            pl.BlockSpec((D, D), lambda b: (0, 0)),
            pl.BlockSpec((D, D), lambda b: (0, 0)),
        ],
        out_specs=[
            pl.BlockSpec((N // 8, D), lambda b: (b, 0)),
            pl.BlockSpec((N // 8, D), lambda b: (b, 0)),
            pl.BlockSpec((N // 8, D), lambda b: (b, 0)),
            pl.BlockSpec((N // 8, 1), lambda b: (b, 0)),
        ],
        out_shape=[
            jax.ShapeDtypeStruct((N, D), jnp.float32),
            jax.ShapeDtypeStruct((N, D), jnp.float32),
            jax.ShapeDtypeStruct((N, D), jnp.float32),
            jax.ShapeDtypeStruct((N, 1), jnp.float32),
        ],
    )(xf, waug, bb, wr, wn)


def _proj2(f, wr, wn):
    return pl.pallas_call(
        _proj2_body,
        grid=(8,),
        in_specs=[
            pl.BlockSpec((N // 8, D), lambda b: (b, 0)),
            pl.BlockSpec((D, D), lambda b: (0, 0)),
            pl.BlockSpec((D, D), lambda b: (0, 0)),
        ],
        out_specs=[
            pl.BlockSpec((N // 8, D), lambda b: (b, 0)),
            pl.BlockSpec((N // 8, D), lambda b: (b, 0)),
            pl.BlockSpec((N // 8, 1), lambda b: (b, 0)),
        ],
        out_shape=[
            jax.ShapeDtypeStruct((N, D), jnp.float32),
            jax.ShapeDtypeStruct((N, D), jnp.float32),
            jax.ShapeDtypeStruct((N, 1), jnp.float32),
        ],
    )(f, wr, wn)


def _knn(feat, sq3):
    return pl.pallas_call(
        _knn_body,
        grid=(NBLK,),
        in_specs=[
            pl.BlockSpec((BM, D), lambda b: (b, 0)),
            pl.BlockSpec((N, D), lambda b: (0, 0)),
            pl.BlockSpec((NT, 1, TCOL), lambda b: (0, 0, 0)),
        ],
        out_specs=pl.BlockSpec((BM, K), lambda b: (b, 0)),
        out_shape=jax.ShapeDtypeStruct((N, K), jnp.int32),
        scratch_shapes=[pltpu.VMEM((NT, BM, TCOL), jnp.float32)],
    )(feat, feat, sq3)


def _agg_sc_body(xn_hbm, xr_hbm, idx_hbm, scale_hbm, shift_hbm, out_hbm,
                 idx_v, rows_a, rows_b, out_v, xr_v, scale_v, shift_v,
                 sem_a, sem_b):
    wid = lax.axis_index("s") * 2 + lax.axis_index("c")
    base = wid * NODES_PER_W
    pltpu.sync_copy(idx_hbm.at[wid], idx_v)
    pltpu.sync_copy(scale_hbm, scale_v)
    pltpu.sync_copy(shift_hbm, shift_v)

    def start(c, buf, sem):
        pltpu.async_copy(xn_hbm.at[idx_v.at[c]], buf, sem)

    def drain(buf, sem):
        pltpu.make_async_copy(
            xn_hbm.at[pl.ds(0, IDX_PER_CHUNK)], buf, sem).wait()

    def compute(c, buf):
        row0 = base + c * NODES_PER_CHUNK
        pltpu.sync_copy(xr_hbm.at[pl.ds(row0, NODES_PER_CHUNK)], xr_v)
        for n in range(NODES_PER_CHUNK):
            def rbody(r, accs):
                return tuple(accs[dd] + buf[n * K + r, pl.ds(dd * 16, 16)]
                             for dd in range(16))
            acc0 = tuple(buf[n * K, pl.ds(dd * 16, 16)] for dd in range(16))
            accs = lax.fori_loop(1, K, rbody, acc0)
            for dd in range(16):
                sl = pl.ds(dd * 16, 16)
                out_v[n, sl] = ((xr_v[n, sl] + accs[dd]) * scale_v[0, sl]
                                + shift_v[0, sl])
        pltpu.sync_copy(out_v, out_hbm.at[pl.ds(row0, NODES_PER_CHUNK)])

    start(0, rows_a, sem_a)

    def pair_body(i, _):
        c0 = 2 * i
        start(c0 + 1, rows_b, sem_b)
        drain(rows_a, sem_a)
        compute(c0, rows_a)

        @pl.when(c0 + 2 < CHUNKS_PER_W)
        def _():
            start(c0 + 2, rows_a, sem_a)

        drain(rows_b, sem_b)
        compute(c0 + 1, rows_b)
        return 0

    lax.fori_loop(0, CHUNKS_PER_W // 2, pair_body, 0)


def _agg_sc(xn, xr, idx3, scale, shift):
    mesh = plsc.VectorSubcoreMesh(core_axis_name="c", subcore_axis_name="s")
    k = functools.partial(
        pl.kernel,
        out_type=jax.ShapeDtypeStruct((N, D), jnp.float32),
        mesh=mesh,
        scratch_types=[
            pltpu.VMEM((CHUNKS_PER_W, IDX_PER_CHUNK), jnp.int32),
            pltpu.VMEM((IDX_PER_CHUNK, D), jnp.float32),
            pltpu.VMEM((IDX_PER_CHUNK, D), jnp.float32),
            pltpu.VMEM((NODES_PER_CHUNK, D), jnp.float32),
            pltpu.VMEM((NODES_PER_CHUNK, D), jnp.float32),
            pltpu.VMEM((1, D), jnp.float32),
            pltpu.VMEM((1, D), jnp.float32),
            pltpu.SemaphoreType.DMA,
            pltpu.SemaphoreType.DMA,
        ],
    )(_agg_sc_body)
    return k(xn, xr, idx3, scale, shift)


def kernel(x, W_base, b_base, W1_root, W1_nbr, b1, bn_gamma, bn_beta,
           W2_root, W2_nbr, b2):
    waug = jnp.repeat(jnp.transpose(W_base)[:, None, :] / 16.0, 16,
                      axis=1).reshape(C_IN * 16, D)
    xf = x.reshape(N, C_IN * 16)
    bb = b_base.reshape(1, D)
    gp = bn_gamma / jnp.sqrt(1.0 + EPS)
    scale1 = gp.reshape(1, D)
    shift1 = (b1 * gp + bn_beta).reshape(1, D)
    scale2 = jnp.ones((1, D), jnp.float32)
    shift2 = b2.reshape(1, D)

    feat, xr1, xn1, sq1 = _proj1(xf, waug, bb, W1_root, W1_nbr)
    idx1 = _knn(feat, sq1.reshape(NT, 1, TCOL))
    f2 = _agg_sc(xn1, xr1, idx1.reshape(NW, CHUNKS_PER_W, IDX_PER_CHUNK),
                 scale1, shift1)
    xr2, xn2, sq2 = _proj2(f2, W2_root, W2_nbr)
    idx2 = _knn(f2, sq2.reshape(NT, 1, TCOL))
    out = _agg_sc(xn2, xr2, idx2.reshape(NW, CHUNKS_PER_W, IDX_PER_CHUNK),
                  scale2, shift2)
    return out


# speedup vs baseline: 1.3807x; 1.0344x over previous
import functools

import jax
import jax.numpy as jnp
from jax import lax
from jax.experimental import pallas as pl
from jax.experimental.pallas import tpu as pltpu
from jax.experimental.pallas import tpu_sc as plsc

N = 4096
C_IN = 128
D = 256
K = 32
EPS = 1e-5

BM = 1024
NT = 8
TCOL = N // NT
NBLK = N // BM

NW = 32
NODES_PER_W = N // NW
NODES_PER_CHUNK = 4
CHUNKS_PER_W = NODES_PER_W // NODES_PER_CHUNK
IDX_PER_CHUNK = NODES_PER_CHUNK * K


def _proj1_body(xf_ref, waug_ref, bb_ref, wr_ref, wn_ref,
                feat_ref, xr_ref, xn_ref, sq_ref):
    f = jnp.dot(xf_ref[...], waug_ref[...],
                preferred_element_type=jnp.float32) + bb_ref[...]
    feat_ref[...] = f
    xr_ref[...] = lax.dot_general(f, wr_ref[...], (((1,), (1,)), ((), ())),
                                  preferred_element_type=jnp.float32)
    xn_ref[...] = lax.dot_general(f, wn_ref[...], (((1,), (1,)), ((), ())),
                                  preferred_element_type=jnp.float32)
    sq_ref[...] = jnp.sum(f * f, axis=1, keepdims=True)


def _proj2_body(f_ref, wr_ref, wn_ref, xr_ref, xn_ref, sq_ref):
    f = f_ref[...]
    xr_ref[...] = lax.dot_general(f, wr_ref[...], (((1,), (1,)), ((), ())),
                                  preferred_element_type=jnp.float32)
    xn_ref[...] = lax.dot_general(f, wn_ref[...], (((1,), (1,)), ((), ())),
                                  preferred_element_type=jnp.float32)
    sq_ref[...] = jnp.sum(f * f, axis=1, keepdims=True)


def _knn_body(fb_ref, ff_ref, sqr_ref, idx_ref, s_ref):
    fb = fb_ref[...]
    for c in range(NT):
        g = lax.dot_general(fb, ff_ref[c * TCOL:(c + 1) * TCOL, :],
                            (((1,), (1,)), ((), ())),
                            preferred_element_type=jnp.float32)
        s_ref[c] = 2.0 * g - sqr_ref[c]

    kiota = lax.broadcasted_iota(jnp.int32, (1, K), 1)
    tiota = lax.broadcasted_iota(jnp.int32, (1, TCOL), 1)
    neg_inf = jnp.float32(-jnp.inf)

    def lexmerge(a, b):
        m1, j1 = a
        m2, j2 = b
        m = jnp.maximum(m1, m2)
        j = jnp.where(m1 > m2, j1,
                      jnp.where(m2 > m1, j2, jnp.minimum(j1, j2)))
        return (m, j)

    def step(t, carry):
        J, jprev = carry
        pairs = []
        for c in range(NT):
            tile = s_ref[c]
            ii = tiota + c * TCOL
            masked = jnp.where(ii == jprev, neg_inf, tile)
            s_ref[c] = masked
            tmax = jnp.max(masked, axis=1, keepdims=True)
            tj = jnp.min(jnp.where(masked == tmax, ii, N), axis=1,
                         keepdims=True)
            pairs.append((tmax, tj))
        while len(pairs) > 1:
            pairs = [lexmerge(pairs[i], pairs[i + 1])
                     for i in range(0, len(pairs), 2)]
        _, j = pairs[0]
        return (jnp.where(kiota == t, j, J), j)

    J, _ = lax.fori_loop(
        0, K, step,
        (jnp.zeros((BM, K), dtype=jnp.int32),
         jnp.full((BM, 1), -1, dtype=jnp.int32)))
    idx_ref[...] = J


def _proj1(xf, waug, bb, wr, wn):
    return pl.pallas_call(
        _proj1_body,
        grid=(8,),
        in_specs=[
            pl.BlockSpec((N // 8, C_IN * 16), lambda b: (b, 0)),
            pl.BlockSpec((C_IN * 16, D), lambda b: (0, 0)),
            pl.BlockSpec((1, D), lambda b: (0, 0)),
            pl.BlockSpec((D, D), lambda b: (0, 0)),
            pl.BlockSpec((D, D), lambda b: (0, 0)),
        ],
        out_specs=[
            pl.BlockSpec((N // 8, D), lambda b: (b, 0)),
            pl.BlockSpec((N // 8, D), lambda b: (b, 0)),
            pl.BlockSpec((N // 8, D), lambda b: (b, 0)),
            pl.BlockSpec((N // 8, 1), lambda b: (b, 0)),
        ],
        out_shape=[
            jax.ShapeDtypeStruct((N, D), jnp.float32),
            jax.ShapeDtypeStruct((N, D), jnp.float32),
            jax.ShapeDtypeStruct((N, D), jnp.float32),
            jax.ShapeDtypeStruct((N, 1), jnp.float32),
        ],
    )(xf, waug, bb, wr, wn)


def _proj2(f, wr, wn):
    return pl.pallas_call(
        _proj2_body,
        grid=(8,),
        in_specs=[
            pl.BlockSpec((N // 8, D), lambda b: (b, 0)),
            pl.BlockSpec((D, D), lambda b: (0, 0)),
            pl.BlockSpec((D, D), lambda b: (0, 0)),
        ],
        out_specs=[
            pl.BlockSpec((N // 8, D), lambda b: (b, 0)),
            pl.BlockSpec((N // 8, D), lambda b: (b, 0)),
            pl.BlockSpec((N // 8, 1), lambda b: (b, 0)),
        ],
        out_shape=[
            jax.ShapeDtypeStruct((N, D), jnp.float32),
            jax.ShapeDtypeStruct((N, D), jnp.float32),
            jax.ShapeDtypeStruct((N, 1), jnp.float32),
        ],
    )(f, wr, wn)


def _knn(feat, sq3):
    return pl.pallas_call(
        _knn_body,
        grid=(NBLK,),
        in_specs=[
            pl.BlockSpec((BM, D), lambda b: (b, 0)),
            pl.BlockSpec((N, D), lambda b: (0, 0)),
            pl.BlockSpec((NT, 1, TCOL), lambda b: (0, 0, 0)),
        ],
        out_specs=pl.BlockSpec((BM, K), lambda b: (b, 0)),
        out_shape=jax.ShapeDtypeStruct((N, K), jnp.int32),
        scratch_shapes=[pltpu.VMEM((NT, BM, TCOL), jnp.float32)],
    )(feat, feat, sq3)


def _make_agg_body(nodes_per_w, base_off):
  chunks_per_w = nodes_per_w // NODES_PER_CHUNK

  def _agg_sc_body(xn_hbm, xr_hbm, idx_hbm, scale_hbm, shift_hbm, out_hbm,
                   idx_v, rows_a, rows_b, out_v, xr_v, scale_v, shift_v,
                   sem_a, sem_b):
    wid = lax.axis_index("s") * 2 + lax.axis_index("c")
    base = wid * nodes_per_w
    pltpu.sync_copy(idx_hbm.at[wid], idx_v)
    pltpu.sync_copy(scale_hbm, scale_v)
    pltpu.sync_copy(shift_hbm, shift_v)

    def start(c, buf, sem):
        pltpu.async_copy(xn_hbm.at[idx_v.at[c]], buf, sem)

    def drain(buf, sem):
        pltpu.make_async_copy(
            xn_hbm.at[pl.ds(0, IDX_PER_CHUNK)], buf, sem).wait()

    def compute(c, buf):
        row0 = base + c * NODES_PER_CHUNK
        pltpu.sync_copy(xr_hbm.at[pl.ds(base_off + row0, NODES_PER_CHUNK)], xr_v)
        for n in range(NODES_PER_CHUNK):
            def rbody(r, accs):
                return tuple(accs[dd] + buf[n * K + r, pl.ds(dd * 16, 16)]
                             for dd in range(16))
            acc0 = tuple(buf[n * K, pl.ds(dd * 16, 16)] for dd in range(16))
            accs = lax.fori_loop(1, K, rbody, acc0)
            for dd in range(16):
                sl = pl.ds(dd * 16, 16)
                out_v[n, sl] = ((xr_v[n, sl] + accs[dd]) * scale_v[0, sl]
                                + shift_v[0, sl])
        pltpu.sync_copy(out_v, out_hbm.at[pl.ds(row0, NODES_PER_CHUNK)])

    start(0, rows_a, sem_a)

    def pair_body(i, _):
        c0 = 2 * i
        start(c0 + 1, rows_b, sem_b)
        drain(rows_a, sem_a)
        compute(c0, rows_a)

        @pl.when(c0 + 2 < chunks_per_w)
        def _():
            start(c0 + 2, rows_a, sem_a)

        drain(rows_b, sem_b)
        compute(c0 + 1, rows_b)
        return 0

    lax.fori_loop(0, chunks_per_w // 2, pair_body, 0)

  return _agg_sc_body


def _agg_sc(xn, xr, idx3, scale, shift, nsub=N, base_off=0):
    nodes_per_w = nsub // NW
    chunks_per_w = nodes_per_w // NODES_PER_CHUNK
    mesh = plsc.VectorSubcoreMesh(core_axis_name="c", subcore_axis_name="s")
    k = functools.partial(
        pl.kernel,
        out_type=jax.ShapeDtypeStruct((nsub, D), jnp.float32),
        mesh=mesh,
        scratch_types=[
            pltpu.VMEM((chunks_per_w, IDX_PER_CHUNK), jnp.int32),
            pltpu.VMEM((IDX_PER_CHUNK, D), jnp.float32),
            pltpu.VMEM((IDX_PER_CHUNK, D), jnp.float32),
            pltpu.VMEM((NODES_PER_CHUNK, D), jnp.float32),
            pltpu.VMEM((NODES_PER_CHUNK, D), jnp.float32),
            pltpu.VMEM((1, D), jnp.float32),
            pltpu.VMEM((1, D), jnp.float32),
            pltpu.SemaphoreType.DMA,
            pltpu.SemaphoreType.DMA,
        ],
    )(_make_agg_body(nodes_per_w, base_off))
    return k(xn, xr, idx3, scale, shift)


def _knn_half(feat, sq3, half):
    off = half * (NBLK // 2)
    return pl.pallas_call(
        _knn_body,
        grid=(NBLK // 2,),
        in_specs=[
            pl.BlockSpec((BM, D), lambda b, off=off: (b + off, 0)),
            pl.BlockSpec((N, D), lambda b: (0, 0)),
            pl.BlockSpec((NT, 1, TCOL), lambda b: (0, 0, 0)),
        ],
        out_specs=pl.BlockSpec((BM, K), lambda b: (b, 0)),
        out_shape=jax.ShapeDtypeStruct((N // 2, K), jnp.int32),
        scratch_shapes=[pltpu.VMEM((NT, BM, TCOL), jnp.float32)],
    )(feat, feat, sq3)


def kernel(x, W_base, b_base, W1_root, W1_nbr, b1, bn_gamma, bn_beta,
           W2_root, W2_nbr, b2):
    waug = jnp.repeat(jnp.transpose(W_base)[:, None, :] / 16.0, 16,
                      axis=1).reshape(C_IN * 16, D)
    xf = x.reshape(N, C_IN * 16)
    bb = b_base.reshape(1, D)
    gp = bn_gamma / jnp.sqrt(1.0 + EPS)
    scale1 = gp.reshape(1, D)
    shift1 = (b1 * gp + bn_beta).reshape(1, D)
    scale2 = jnp.ones((1, D), jnp.float32)
    shift2 = b2.reshape(1, D)

    half_chunks = (N // 2 // NW) // NODES_PER_CHUNK

    def round_halves(xn, xr, feat_knn, sq3, scale, shift):
        parts = []
        for h in (0, 1):
            idx_h = _knn_half(feat_knn, sq3, h)
            parts.append(_agg_sc(
                xn, xr, idx_h.reshape(NW, half_chunks, IDX_PER_CHUNK),
                scale, shift, nsub=N // 2, base_off=h * (N // 2)))
        return jnp.concatenate(parts, axis=0)

    feat, xr1, xn1, sq1 = _proj1(xf, waug, bb, W1_root, W1_nbr)
    f2 = round_halves(xn1, xr1, feat, sq1.reshape(NT, 1, TCOL),
                      scale1, shift1)
    xr2, xn2, sq2 = _proj2(f2, W2_root, W2_nbr)
    out = round_halves(xn2, xr2, f2, sq2.reshape(NT, 1, TCOL),
                       scale2, shift2)
    return out
